# baseline (speedup 1.0000x reference)
import functools

import jax
import jax.numpy as jnp
from jax import lax
from jax.experimental import pallas as pl
from jax.experimental.pallas import tpu as pltpu

POOL = 64
K = 8
B = 64
SEQ = 197
D = 768

BB = 16
NSTEPS = B // BB


def _l2norm_rows(x):
    sq = jnp.sum(x * x, axis=1, keepdims=True)
    return x * lax.rsqrt(jnp.maximum(sq, 1e-12))


def _fused(x_ref, key_ref, sim_ref, bkn_ref, rs_ref, idx_ref, xmax_ref):
    i = pl.program_id(0)
    xmax_ref[pl.ds(i * BB, BB), :] = jnp.max(x_ref[...], axis=1)

    @pl.when(i == NSTEPS - 1)
    def _routing():
        x_max = xmax_ref[...]
        k_norm = _l2norm_rows(key_ref[...])
        x_norm = _l2norm_rows(x_max)
        sim = lax.dot_general(
            x_norm, k_norm, (((1,), (1,)), ((), ())),
            preferred_element_type=jnp.float32)
        sim_ref[...] = sim

        iota = lax.broadcasted_iota(jnp.int32, (B, POOL), 1)
        work = sim
        total = jnp.float32(0.0)
        for kk in range(K):
            m = jnp.max(work, axis=1, keepdims=True)
            amax = jnp.min(jnp.where(work == m, iota, POOL),
                           axis=1, keepdims=True)
            idx_ref[:, kk:kk + 1] = amax
            onehot = (iota == amax).astype(jnp.float32)
            row = jnp.dot(onehot, k_norm, precision=lax.Precision.HIGHEST,
                          preferred_element_type=jnp.float32)
            bkn_ref[:, kk, :] = row
            total = total + jnp.sum(row * x_norm)
            work = jnp.where(iota == amax, -jnp.inf, work)
        rs_ref[...] = jnp.broadcast_to(total / jnp.float32(B), (1, 1))


@functools.partial(jax.jit, static_argnames=("interpret",))
def kernel(x_embed, prompt_key, interpret=False):
    sim, bkn, rs, idx = pl.pallas_call(
        _fused,
        grid=(NSTEPS,),
        in_specs=[
            pl.BlockSpec((BB, SEQ, D), lambda i: (i, 0, 0)),
            pl.BlockSpec((POOL, D), lambda i: (0, 0)),
        ],
        out_specs=[
            pl.BlockSpec((B, POOL), lambda i: (0, 0)),
            pl.BlockSpec((B, K, D), lambda i: (0, 0, 0)),
            pl.BlockSpec((1, 1), lambda i: (0, 0)),
            pl.BlockSpec((B, K), lambda i: (0, 0)),
        ],
        out_shape=[
            jax.ShapeDtypeStruct((B, POOL), jnp.float32),
            jax.ShapeDtypeStruct((B, K, D), jnp.float32),
            jax.ShapeDtypeStruct((1, 1), jnp.float32),
            jax.ShapeDtypeStruct((B, K), jnp.int32),
        ],
        scratch_shapes=[pltpu.VMEM((B, D), jnp.float32)],
        compiler_params=pltpu.CompilerParams(
            dimension_semantics=("arbitrary",)),
        interpret=interpret,
    )(x_embed, prompt_key)
    return sim, bkn, rs[0, 0], idx


# speedup vs baseline: 1.0054x; 1.0054x over previous
import functools

import jax
import jax.numpy as jnp
from jax import lax
from jax.experimental import pallas as pl
from jax.experimental.pallas import tpu as pltpu

POOL = 64
K = 8
B = 64
SEQ = 197
D = 768

NBUF = 8
BB = B // NBUF


def _l2norm_rows(x):
    sq = jnp.sum(x * x, axis=1, keepdims=True)
    return x * lax.rsqrt(jnp.maximum(sq, 1e-12))


def _body(x_hbm, key_ref, sim_ref, bkn_ref, rs_ref, idx_ref,
          xmax_ref, sems, *bufs):
    copies = [
        pltpu.make_async_copy(x_hbm.at[pl.ds(s * BB, BB)], bufs[s],
                              sems.at[s])
        for s in range(NBUF)
    ]
    for c in copies:
        c.start()
    for s in range(NBUF):
        copies[s].wait()
        xmax_ref[pl.ds(s * BB, BB), :] = jnp.max(bufs[s][...], axis=1)

    x_max = xmax_ref[...]
    k_norm = _l2norm_rows(key_ref[...])
    x_norm = _l2norm_rows(x_max)
    sim = lax.dot_general(
        x_norm, k_norm, (((1,), (1,)), ((), ())),
        preferred_element_type=jnp.float32)
    sim_ref[...] = sim

    iota = lax.broadcasted_iota(jnp.int32, (B, POOL), 1)
    work = sim
    total = jnp.float32(0.0)
    for kk in range(K):
        m = jnp.max(work, axis=1, keepdims=True)
        amax = jnp.min(jnp.where(work == m, iota, POOL),
                       axis=1, keepdims=True)
        idx_ref[:, kk:kk + 1] = amax
        onehot = (iota == amax).astype(jnp.float32)
        row = jnp.dot(onehot, k_norm, precision=lax.Precision.HIGHEST,
                      preferred_element_type=jnp.float32)
        bkn_ref[:, kk, :] = row
        total = total + jnp.sum(row * x_norm)
        work = jnp.where(iota == amax, -jnp.inf, work)
    rs_ref[...] = jnp.broadcast_to(total / jnp.float32(B), (1, 1))


@functools.partial(jax.jit, static_argnames=("interpret",))
def kernel(x_embed, prompt_key, interpret=False):
    sim, bkn, rs, idx = pl.pallas_call(
        _body,
        in_specs=[
            pl.BlockSpec(memory_space=pl.ANY),
            pl.BlockSpec((POOL, D), lambda: (0, 0)),
        ],
        out_specs=[
            pl.BlockSpec((B, POOL), lambda: (0, 0)),
            pl.BlockSpec((B, K, D), lambda: (0, 0, 0)),
            pl.BlockSpec((1, 1), lambda: (0, 0)),
            pl.BlockSpec((B, K), lambda: (0, 0)),
        ],
        out_shape=[
            jax.ShapeDtypeStruct((B, POOL), jnp.float32),
            jax.ShapeDtypeStruct((B, K, D), jnp.float32),
            jax.ShapeDtypeStruct((1, 1), jnp.float32),
            jax.ShapeDtypeStruct((B, K), jnp.int32),
        ],
        scratch_shapes=(
            [pltpu.VMEM((B, D), jnp.float32),
             pltpu.SemaphoreType.DMA((NBUF,))]
            + [pltpu.VMEM((BB, SEQ, D), jnp.float32) for _ in range(NBUF)]
        ),
        interpret=interpret,
    )(x_embed, prompt_key)
    return sim, bkn, rs[0, 0], idx
